# baseline (speedup 1.0000x reference)
import functools

import jax
import jax.numpy as jnp
from jax import lax
from jax.experimental import pallas as pl
from jax.experimental.pallas import tpu as pltpu
from jax.experimental.pallas import tpu_sc as plsc

_B = 16384
_D = 16
_NC = 2
_NS = 16
_NW = _NC * _NS
_BPW = _B // _NW
_CHUNK = 128
_NCH = _BPW // _CHUNK
_GROUPS = _BPW // 16

_mesh = plsc.VectorSubcoreMesh(core_axis_name="c", subcore_axis_name="s")


@functools.partial(
    pl.kernel,
    out_type=jax.ShapeDtypeStruct((_B,), jnp.float32),
    mesh=_mesh,
    scratch_types=[
        pltpu.VMEM((_BPW,), jnp.int32),
        pltpu.VMEM((_BPW,), jnp.int32),
        pltpu.VMEM((_BPW, _D), jnp.float32),
        pltpu.VMEM((_BPW, _D), jnp.float32),
        pltpu.VMEM((_BPW,), jnp.float32),
        pltpu.SemaphoreType.DMA,
    ],
    compiler_params=pltpu.CompilerParams(use_tc_tiling_on_sc=False,
                                         needs_layout_passes=False),
)
def _mf_sc(list_idx, item_idx, list_tab, item_tab, out,
           idxl_v, idxi_v, rows_l, rows_i, out_v, sem):
    wid = lax.axis_index("s") * _NC + lax.axis_index("c")
    base = wid * _BPW

    pltpu.sync_copy(list_idx.at[pl.ds(base, _BPW)], idxl_v)
    pltpu.sync_copy(item_idx.at[pl.ds(base, _BPW)], idxi_v)

    copies = []
    for c in range(_NCH):
        sl = pl.ds(c * _CHUNK, _CHUNK)
        copies.append(pltpu.async_copy(list_tab.at[idxl_v.at[sl]], rows_l.at[sl], sem))
        copies.append(pltpu.async_copy(item_tab.at[idxi_v.at[sl]], rows_i.at[sl], sem))
    for cp in copies:
        cp.wait()

    lane = lax.iota(jnp.int32, 16)

    def group(g, carry):
        rows = g * 16 + lane
        acc = jnp.zeros((16,), jnp.float32)
        for t in range(_D):
            col = (lane + t) & (_D - 1)
            lv = plsc.load_gather(rows_l, [rows, col])
            iv = plsc.load_gather(rows_i, [rows, col])
            acc = acc + lv * iv
        out_v[pl.ds(g * 16, 16)] = 1.0 / (1.0 + jnp.exp(-acc))
        return carry

    lax.fori_loop(0, _GROUPS, group, 0)
    pltpu.sync_copy(out_v, out.at[pl.ds(base, _BPW)])


def kernel(user_indices, list_indices, item_indices,
           user_table, list_table, item_table):
    del user_indices, user_table
    return _mf_sc(list_indices.astype(jnp.int32),
                  item_indices.astype(jnp.int32),
                  list_table, item_table)
